# Initial kernel scaffold; baseline (speedup 1.0000x reference)
import jax
import jax.numpy as jnp
from jax.experimental import pallas as pl


def kernel(u, v, batch):
    raise NotImplementedError("write your pallas kernel here")


# speedup vs baseline: 4.0962x; 4.0962x over previous
import functools

import jax
import jax.numpy as jnp
from jax import lax
from jax.experimental import pallas as pl
from jax.experimental.pallas import tpu as pltpu
from jax.experimental.pallas import tpu_sc as plsc

N_SEGMENTS = 10000
N_ELEMS = 320000
D = 128

NC = 2
NS = 16
HALF = N_SEGMENTS // 2
ROWS_PER_TILE = 320
ACC_ROWS = HALF + 8
CB = 128

@functools.cache
def _build_kernel():
    mesh = plsc.VectorSubcoreMesh(core_axis_name="c", subcore_axis_name="s",
                                  num_cores=NC, num_subcores=NS)

    @functools.partial(
        pl.kernel,
        out_type=jax.ShapeDtypeStruct((N_SEGMENTS, D), jnp.float32),
        mesh=mesh,
        scratch_types=[
            pltpu.VMEM_SHARED((ACC_ROWS, D), jnp.float32),
            pltpu.VMEM((CB, D), jnp.float32),
            pltpu.VMEM((CB,), jnp.int32),
            pltpu.VMEM((16,), jnp.int32),
        ],
    )
    def _scatter_add_kernel(u_hbm, v_hbm, batch_hbm, split_hbm, out_hbm,
                            acc, vbuf, ibuf, split_v):
        _kernel_body(u_hbm, v_hbm, batch_hbm, split_hbm, out_hbm,
                     acc, vbuf, ibuf, split_v)

    return _scatter_add_kernel


def _kernel_body(u_hbm, v_hbm, batch_hbm, split_hbm, out_hbm,
                 acc, vbuf, ibuf, split_v):
    c = lax.axis_index("c")
    t = lax.axis_index("s")

    pltpu.sync_copy(split_hbm, split_v)
    split = split_v[...][0]

    r0 = jnp.minimum(t * ROWS_PER_TILE, HALF - ROWS_PER_TILE)
    pltpu.sync_copy(u_hbm.at[pl.ds(c * HALF + r0, ROWS_PER_TILE), :],
                    acc.at[pl.ds(r0, ROWS_PER_TILE), :])
    plsc.subcore_barrier()

    base = jnp.where(c == 0, 0, split)
    length = jnp.where(c == 0, split, N_ELEMS - split)
    chunk = (length + NS - 1) // NS
    lo = base + jnp.minimum(t * chunk, length)
    hi = base + jnp.minimum((t + 1) * chunk, length)
    alo = (lo // 8) * 8
    num_chunks = jnp.maximum(0, (hi - alo + CB - 1) // CB)
    seg_base = c * HALF

    def body(k, carry):
        s_orig = alo + k * CB
        s = jnp.minimum(s_orig, N_ELEMS - CB)
        pltpu.sync_copy(v_hbm.at[pl.ds(s, CB), :], vbuf)
        pltpu.sync_copy(batch_hbm.at[pl.ds(s, CB)], ibuf)
        for j in range(CB // 16):
            idx = ibuf[pl.ds(j * 16, 16)]
            g = s + j * 16 + lax.iota(jnp.int32, 16)
            valid = (g >= lo) & (g < hi) & (g >= s_orig)
            ibuf[pl.ds(j * 16, 16)] = jnp.where(valid, idx - seg_base, HALF)
        pltpu.sync_copy(vbuf, acc.at[ibuf], add=True)
        return carry

    lax.fori_loop(0, num_chunks, body, 0)
    plsc.subcore_barrier()

    pltpu.sync_copy(acc.at[pl.ds(r0, ROWS_PER_TILE), :],
                    out_hbm.at[pl.ds(c * HALF + r0, ROWS_PER_TILE), :])


def kernel(u, v, batch):
    batch = batch.astype(jnp.int32)
    split = jnp.searchsorted(batch, jnp.int32(HALF), side="left")
    split_arr = jnp.full((16,), split, dtype=jnp.int32)
    return _build_kernel()(u, v, batch, split_arr)


# speedup vs baseline: 6.6312x; 1.6189x over previous
import functools

import jax
import jax.numpy as jnp
from jax import lax
from jax.experimental import pallas as pl
from jax.experimental.pallas import tpu as pltpu
from jax.experimental.pallas import tpu_sc as plsc

N_SEGMENTS = 10000
N_ELEMS = 320000
D = 128

NC = 2
NS = 16
HALF = N_SEGMENTS // 2
ROWS_PER_TILE = 320
ACC_ROWS = HALF + 8
CB = 128


@functools.cache
def _build_kernel():
    mesh = plsc.VectorSubcoreMesh(core_axis_name="c", subcore_axis_name="s",
                                  num_cores=NC, num_subcores=NS)

    @functools.partial(
        pl.kernel,
        out_type=jax.ShapeDtypeStruct((N_SEGMENTS, D), jnp.float32),
        mesh=mesh,
        scratch_types=[
            pltpu.VMEM_SHARED((ACC_ROWS, D), jnp.float32),
            pltpu.VMEM((CB, D), jnp.float32),
            pltpu.VMEM((CB, D), jnp.float32),
            pltpu.VMEM((CB,), jnp.int32),
            pltpu.VMEM((CB,), jnp.int32),
            pltpu.VMEM((16,), jnp.int32),
            pltpu.SemaphoreType.DMA,
            pltpu.SemaphoreType.DMA,
        ],
    )
    def _scatter_add_kernel(u_hbm, v_hbm, batch_hbm, split_hbm, out_hbm,
                            acc, vbuf0, vbuf1, ibuf0, ibuf1, split_v,
                            sem0, sem1):
        c = lax.axis_index("c")
        t = lax.axis_index("s")

        pltpu.sync_copy(split_hbm, split_v)
        split = split_v[...][0]

        r0 = jnp.minimum(t * ROWS_PER_TILE, HALF - ROWS_PER_TILE)
        pltpu.sync_copy(u_hbm.at[pl.ds(c * HALF + r0, ROWS_PER_TILE), :],
                        acc.at[pl.ds(r0, ROWS_PER_TILE), :])
        plsc.subcore_barrier()

        base = jnp.where(c == 0, 0, split)
        length = jnp.where(c == 0, split, N_ELEMS - split)
        chunk = (length + NS - 1) // NS
        lo = base + jnp.minimum(t * chunk, length)
        hi = base + jnp.minimum((t + 1) * chunk, length)
        alo = (lo // 8) * 8
        num_chunks = jnp.maximum(0, (hi - alo + CB - 1) // CB)
        seg_base = c * HALF

        def chunk_start(k):
            s_orig = alo + k * CB
            return jnp.minimum(s_orig, N_ELEMS - CB), s_orig

        def issue(k, vbuf, ibuf, sem):
            s, _ = chunk_start(k)
            pltpu.async_copy(v_hbm.at[pl.ds(s, CB), :], vbuf, sem)
            pltpu.async_copy(batch_hbm.at[pl.ds(s, CB)], ibuf, sem)

        def drain(vbuf, ibuf, sem):
            pltpu.make_async_copy(v_hbm.at[pl.ds(0, CB), :], vbuf, sem).wait()
            pltpu.make_async_copy(batch_hbm.at[pl.ds(0, CB)], ibuf, sem).wait()

        def process(k, vbuf, ibuf):
            s, s_orig = chunk_start(k)
            for j in range(CB // 16):
                idx = ibuf[pl.ds(j * 16, 16)]
                g = s + j * 16 + lax.iota(jnp.int32, 16)
                valid = (g >= lo) & (g < hi) & (g >= s_orig)
                ibuf[pl.ds(j * 16, 16)] = jnp.where(valid, idx - seg_base,
                                                    HALF)
            pltpu.sync_copy(vbuf, acc.at[ibuf], add=True)

        @pl.when(num_chunks > 0)
        def _():
            issue(0, vbuf0, ibuf0, sem0)

        def body(kk, carry):
            k0 = kk * 2
            k1 = k0 + 1

            @pl.when(k1 < num_chunks)
            def _():
                issue(k1, vbuf1, ibuf1, sem1)

            drain(vbuf0, ibuf0, sem0)
            process(k0, vbuf0, ibuf0)

            @pl.when(k0 + 2 < num_chunks)
            def _():
                issue(k0 + 2, vbuf0, ibuf0, sem0)

            @pl.when(k1 < num_chunks)
            def _():
                drain(vbuf1, ibuf1, sem1)
                process(k1, vbuf1, ibuf1)

            return carry

        lax.fori_loop(0, (num_chunks + 1) // 2, body, 0)
        plsc.subcore_barrier()

        pltpu.sync_copy(acc.at[pl.ds(r0, ROWS_PER_TILE), :],
                        out_hbm.at[pl.ds(c * HALF + r0, ROWS_PER_TILE), :])

    return _scatter_add_kernel


def kernel(u, v, batch):
    batch = batch.astype(jnp.int32)
    split = jnp.searchsorted(batch, jnp.int32(HALF), side="left")
    split_arr = jnp.full((16,), split, dtype=jnp.int32)
    return _build_kernel()(u, v, batch, split_arr)
